# baseline (speedup 1.0000x reference)
import functools

import jax
import jax.numpy as jnp
from jax import lax
from jax.experimental import pallas as pl
from jax.experimental.pallas import tpu as pltpu
from jax.experimental.pallas import tpu_sc as plsc

VOCAB = 1000000
MAXLEN = 200
EMBED = 64
BATCH = 1024
SEQ = 200

LANES = 16
NW = 32
B_TOTAL = BATCH * SEQ
B_PER_W = B_TOTAL // NW
CHUNK = 128
N_CHUNKS = B_PER_W // CHUNK


def _make_kernel():
  mesh = plsc.VectorSubcoreMesh(core_axis_name="c", subcore_axis_name="s")

  @functools.partial(
      pl.kernel,
      mesh=mesh,
      compiler_params=pltpu.CompilerParams(use_tc_tiling_on_sc=False),
      out_type=jax.ShapeDtypeStruct((B_TOTAL, EMBED), jnp.float32),
      scratch_types=[
          pltpu.VMEM((B_PER_W,), jnp.int32),
          pltpu.VMEM((2 * MAXLEN, EMBED), jnp.float32),
          pltpu.VMEM((CHUNK, EMBED), jnp.float32),
          pltpu.SemaphoreType.DMA,
      ],
  )
  def embed(x_hbm, tok_hbm, pos_hbm, out_hbm, idx_v, pos2_v, rows_v, sem):
    wid = lax.axis_index("s") * 2 + lax.axis_index("c")
    base = wid * B_PER_W
    pltpu.sync_copy(x_hbm.at[pl.ds(base, B_PER_W)], idx_v)
    pltpu.sync_copy(pos_hbm, pos2_v.at[pl.ds(0, MAXLEN)])
    pltpu.sync_copy(pos_hbm, pos2_v.at[pl.ds(MAXLEN, MAXLEN)])

    def chunk_body(c, carry):
      pltpu.async_copy(
          tok_hbm.at[idx_v.at[pl.ds(c * CHUNK, CHUNK)]], rows_v, sem
      ).wait()
      s0 = lax.rem(c * CHUNK, MAXLEN)

      def row_body(r, rcarry):
        for d in range(EMBED // LANES):
          sl = pl.ds(d * LANES, LANES)
          plsc.addupdate(rows_v.at[r, sl], pos2_v[s0 + r, sl])
        return rcarry

      lax.fori_loop(0, CHUNK, row_body, 0, unroll=4)
      pltpu.sync_copy(rows_v, out_hbm.at[pl.ds(base + c * CHUNK, CHUNK)])
      return carry

    lax.fori_loop(0, N_CHUNKS, chunk_body, 0)

  return embed


_embed = _make_kernel()


def kernel(x, token_table, pos_table):
  x_flat = x.reshape(-1).astype(jnp.int32)
  out = _embed(x_flat, token_table, pos_table)
  return out.reshape(BATCH, SEQ, EMBED)


# speedup vs baseline: 1.0566x; 1.0566x over previous
import functools

import jax
import jax.numpy as jnp
from jax import lax
from jax.experimental import pallas as pl
from jax.experimental.pallas import tpu as pltpu
from jax.experimental.pallas import tpu_sc as plsc

VOCAB = 1000000
MAXLEN = 200
EMBED = 64
BATCH = 1024
SEQ = 200

LANES = 16
NW = 32
B_TOTAL = BATCH * SEQ
B_PER_W = B_TOTAL // NW
CHUNK = 128
N_CHUNKS = B_PER_W // CHUNK
NBUF = 5
N_ROUNDS = N_CHUNKS // NBUF


def _make_kernel():
  mesh = plsc.VectorSubcoreMesh(core_axis_name="c", subcore_axis_name="s")

  @functools.partial(
      pl.kernel,
      mesh=mesh,
      compiler_params=pltpu.CompilerParams(use_tc_tiling_on_sc=False),
      out_type=jax.ShapeDtypeStruct((B_TOTAL, EMBED), jnp.float32),
      scratch_types=[
          pltpu.VMEM((B_PER_W,), jnp.int32),
          pltpu.VMEM((2 * MAXLEN, EMBED), jnp.float32),
          [pltpu.VMEM((CHUNK, EMBED), jnp.float32) for _ in range(NBUF)],
          [pltpu.SemaphoreType.DMA for _ in range(NBUF)],
          [pltpu.SemaphoreType.DMA for _ in range(NBUF)],
      ],
  )
  def embed(x_hbm, tok_hbm, pos_hbm, out_hbm, idx_v, pos2_v, rows, gsem, ssem):
    wid = lax.axis_index("s") * 2 + lax.axis_index("c")
    base = wid * B_PER_W
    pltpu.sync_copy(x_hbm.at[pl.ds(base, B_PER_W)], idx_v)
    pltpu.sync_copy(pos_hbm, pos2_v.at[pl.ds(0, MAXLEN)])
    pltpu.sync_copy(pos_hbm, pos2_v.at[pl.ds(MAXLEN, MAXLEN)])

    def gather_start(b, c):
      pltpu.async_copy(
          tok_hbm.at[idx_v.at[pl.ds(c * CHUNK, CHUNK)]], rows[b], gsem[b]
      )

    def gather_wait(b):
      pltpu.make_async_copy(tok_hbm.at[pl.ds(0, CHUNK)], rows[b], gsem[b]).wait()

    def store_start(b, c):
      pltpu.async_copy(rows[b], out_hbm.at[pl.ds(base + c * CHUNK, CHUNK)],
                       ssem[b])

    def store_wait(b, c):
      pltpu.make_async_copy(
          rows[b], out_hbm.at[pl.ds(base + c * CHUNK, CHUNK)], ssem[b]
      ).wait()

    def add_pos(b, c):
      s0 = lax.rem(c * CHUNK, MAXLEN)

      def row_body(r, rcarry):
        for d in range(EMBED // LANES):
          sl = pl.ds(d * LANES, LANES)
          plsc.addupdate(rows[b].at[r, sl], pos2_v[s0 + r, sl])
        return rcarry

      lax.fori_loop(0, CHUNK, row_body, 0, unroll=4)

    for b in range(NBUF):
      gather_start(b, b)

    def round_body(i, carry):
      c0 = i * NBUF
      for b in range(NBUF):
        gather_wait(b)
        add_pos(b, c0 + b)
        store_start(b, c0 + b)
      for b in range(NBUF):
        nxt = c0 + b + NBUF

        @pl.when(nxt < N_CHUNKS)
        def _():
          store_wait(b, c0 + b)
          gather_start(b, nxt)

      return carry

    lax.fori_loop(0, N_ROUNDS, round_body, 0)
    for b in range(NBUF):
      store_wait(b, N_CHUNKS - NBUF + b)

  return embed


_embed = _make_kernel()


def kernel(x, token_table, pos_table):
  x_flat = x.reshape(-1).astype(jnp.int32)
  out = _embed(x_flat, token_table, pos_table)
  return out.reshape(BATCH, SEQ, EMBED)


# speedup vs baseline: 1.1508x; 1.0891x over previous
import functools

import jax
import jax.numpy as jnp
from jax import lax
from jax.experimental import pallas as pl
from jax.experimental.pallas import tpu as pltpu
from jax.experimental.pallas import tpu_sc as plsc

VOCAB = 1000000
MAXLEN = 200
EMBED = 64
BATCH = 1024
SEQ = 200

LANES = 16
NW = 32
SEQ_PER_W = BATCH // NW
CHUNK0 = 128
CHUNK1 = SEQ - CHUNK0
NBUF = 4
N_CHUNKS = 2 * SEQ_PER_W
N_ROUNDS = N_CHUNKS // NBUF


def _chunk_geom(c):
  return c // 2, (c % 2) * CHUNK0, CHUNK1 if c % 2 else CHUNK0


def _make_kernel():
  mesh = plsc.VectorSubcoreMesh(core_axis_name="c", subcore_axis_name="s")

  @functools.partial(
      pl.kernel,
      mesh=mesh,
      compiler_params=pltpu.CompilerParams(use_tc_tiling_on_sc=False),
      out_type=jax.ShapeDtypeStruct((BATCH, SEQ, EMBED), jnp.float32),
      scratch_types=[
          pltpu.VMEM((SEQ_PER_W, SEQ), jnp.int32),
          pltpu.VMEM((MAXLEN, EMBED), jnp.float32),
          [pltpu.VMEM((CHUNK1 if b % 2 else CHUNK0, EMBED), jnp.float32)
           for b in range(NBUF)],
          [pltpu.SemaphoreType.DMA for _ in range(NBUF)],
          [pltpu.SemaphoreType.DMA for _ in range(NBUF)],
      ],
  )
  def embed(x_hbm, tok_hbm, pos_hbm, out_hbm, idx_v, pos_v, rows, gsem, ssem):
    wid = lax.axis_index("s") * 2 + lax.axis_index("c")
    seq_base = wid * SEQ_PER_W
    pltpu.sync_copy(x_hbm.at[pl.ds(seq_base, SEQ_PER_W), :], idx_v)
    pltpu.sync_copy(pos_hbm, pos_v)

    def gather_start(b, seq, s0, ln):
      pltpu.async_copy(
          tok_hbm.at[idx_v.at[seq, pl.ds(s0, ln)]], rows[b], gsem[b]
      )

    def gather_wait(b, ln):
      pltpu.make_async_copy(tok_hbm.at[pl.ds(0, ln)], rows[b], gsem[b]).wait()

    def store_start(b, seq, s0, ln):
      pltpu.async_copy(
          rows[b], out_hbm.at[seq_base + seq, pl.ds(s0, ln), :], ssem[b]
      )

    def store_wait(b, seq, s0, ln):
      pltpu.make_async_copy(
          rows[b], out_hbm.at[seq_base + seq, pl.ds(s0, ln), :], ssem[b]
      ).wait()

    def add_pos(b, s0, ln):
      def row_body(r, rcarry):
        for d in range(EMBED // LANES):
          sl = pl.ds(d * LANES, LANES)
          plsc.addupdate(rows[b].at[r, sl], pos_v[s0 + r, sl])
        return rcarry

      lax.fori_loop(0, ln, row_body, 0, unroll=4)

    for b in range(NBUF):
      seq, s0, ln = _chunk_geom(b)
      gather_start(b, seq, s0, ln)

    def round_body(i, carry):
      c0 = i * NBUF
      for b in range(NBUF):
        s0 = (b % 2) * CHUNK0
        ln = CHUNK1 if b % 2 else CHUNK0
        seq = (c0 + b) // 2
        gather_wait(b, ln)
        add_pos(b, s0, ln)
        store_start(b, seq, s0, ln)
      for b in range(NBUF):
        s0 = (b % 2) * CHUNK0
        ln = CHUNK1 if b % 2 else CHUNK0
        seq = (c0 + b) // 2

        @pl.when(c0 + b + NBUF < N_CHUNKS)
        def _():
          store_wait(b, seq, s0, ln)
          gather_start(b, (c0 + b + NBUF) // 2, s0, ln)

      return carry

    lax.fori_loop(0, N_ROUNDS, round_body, 0)
    for b in range(NBUF):
      seq, s0, ln = _chunk_geom(N_CHUNKS - NBUF + b)
      store_wait(b, seq, s0, ln)

  return embed


_embed = _make_kernel()


def kernel(x, token_table, pos_table):
  return _embed(x.astype(jnp.int32), token_table, pos_table)
